# baseline (speedup 1.0000x reference)
import functools

import jax
import jax.numpy as jnp
from jax import lax
from jax.experimental import pallas as pl
from jax.experimental.pallas import tpu as pltpu
from jax.experimental.pallas import tpu_sc as plsc

G, V, D, P = 8, 64, 64, 32

def _idx_body(x_ref, wpre_ref, bpre_ref, wwp_ref, bwp_ref, idx_ref):
    h = jnp.dot(x_ref[...], wpre_ref[...]) + bpre_ref[...]
    logits = jnp.dot(h, wwp_ref[...]) + bwp_ref[...]
    R = logits.shape[0]
    iota = jax.lax.broadcasted_iota(jnp.int32, (R, V), 1)
    cols = []
    for g in range(G):
        lg = logits[:, g * V:(g + 1) * V]
        mx = jnp.max(lg, axis=1, keepdims=True)
        cand = jnp.where(lg >= mx, iota + g * V, 1 << 30)
        cols.append(jnp.min(cand, axis=1, keepdims=True))
    idx_ref[...] = jnp.concatenate(cols, axis=1)


def _post_body(q_ref, wpost_ref, bpost_ref, out_ref):
    q = q_ref[...].astype(jnp.bfloat16)
    out_ref[...] = (jnp.dot(q, wpost_ref[...],
                            preferred_element_type=jnp.float32)
                    + bpost_ref[...])


def kernel(x, W_pre, b_pre, W_wp, b_wp, codebook, W_post, b_post):
    B, T, IN = x.shape
    BT = B * T
    OUT = W_post.shape[1]
    GV = G * V
    NROWS = BT * G

    R = 2304
    x2 = x.reshape(BT, IN)
    idx = pl.pallas_call(
        _idx_body,
        grid=(BT // R,),
        in_specs=[
            pl.BlockSpec((R, IN), lambda i: (i, 0)),
            pl.BlockSpec((IN, P), lambda i: (0, 0)),
            pl.BlockSpec((1, P), lambda i: (0, 0)),
            pl.BlockSpec((P, GV), lambda i: (0, 0)),
            pl.BlockSpec((1, GV), lambda i: (0, 0)),
        ],
        out_specs=pl.BlockSpec((R, G), lambda i: (i, 0)),
        out_shape=jax.ShapeDtypeStruct((BT, G), jnp.int32),
    )(x2, W_pre, b_pre.reshape(1, P), W_wp, b_wp.reshape(1, GV))
    idx_flat = idx.reshape(NROWS)

    NW = 32
    RPW = NROWS // NW
    CH = 128
    NCH = RPW // CH

    cb128 = jnp.pad(codebook, ((0, 0), (0, 128 - D)))

    @functools.partial(
        pl.kernel,
        mesh=plsc.VectorSubcoreMesh(core_axis_name="c", subcore_axis_name="s"),
        out_type=jax.ShapeDtypeStruct((NROWS, 128), jnp.float32),
        scratch_types=[
            pltpu.VMEM((CH,), jnp.int32),
            pltpu.VMEM((CH, 128), jnp.float32),
            pltpu.SemaphoreType.DMA,
        ],
    )
    def _gather(cb_hbm, idx_hbm, out_hbm, idx_v, rows_v, sem):
        wid = lax.axis_index("s") * 2 + lax.axis_index("c")
        base = wid * RPW
        for c in range(NCH):
            off = base + c * CH
            pltpu.sync_copy(idx_hbm.at[pl.ds(off, CH)], idx_v)
            pltpu.async_copy(cb_hbm.at[idx_v], rows_v, sem).wait()
            pltpu.sync_copy(rows_v, out_hbm.at[pl.ds(off, CH)])

    q2 = _gather(cb128, idx_flat)

    q = q2.reshape(BT, G * 128)
    wpost_pad = jnp.pad(W_post.reshape(G, D, OUT), ((0, 0), (0, 128 - D), (0, 0))
                        ).reshape(G * 128, OUT)
    out = pl.pallas_call(
        _post_body,
        grid=(BT // R,),
        in_specs=[
            pl.BlockSpec((R, G * 128), lambda i: (i, 0)),
            pl.BlockSpec((G * 128, OUT), lambda i: (0, 0)),
            pl.BlockSpec((1, OUT), lambda i: (0, 0)),
        ],
        out_specs=pl.BlockSpec((R, OUT), lambda i: (i, 0)),
        out_shape=jax.ShapeDtypeStruct((BT, OUT), jnp.float32),
    )(q, wpost_pad.astype(jnp.bfloat16), b_post.reshape(1, OUT))
    return out.reshape(B, T, OUT)


# speedup vs baseline: 1.0049x; 1.0049x over previous
import functools

import jax
import jax.numpy as jnp
from jax import lax
from jax.experimental import pallas as pl
from jax.experimental.pallas import tpu as pltpu
from jax.experimental.pallas import tpu_sc as plsc

G, V, D, P = 8, 64, 64, 32

def _idx_body(x_ref, wpre_ref, bpre_ref, wwp_ref, bwp_ref, idx_ref):
    h = jnp.dot(x_ref[...], wpre_ref[...]) + bpre_ref[...]
    logits = jnp.dot(h, wwp_ref[...]) + bwp_ref[...]
    R = logits.shape[0]
    iota = jax.lax.broadcasted_iota(jnp.int32, (R, V), 1)
    cols = []
    for g in range(G):
        lg = logits[:, g * V:(g + 1) * V]
        mx = jnp.max(lg, axis=1, keepdims=True)
        cand = jnp.where(lg >= mx, iota + g * V, 1 << 30)
        cols.append(jnp.min(cand, axis=1, keepdims=True))
    idx_ref[...] = jnp.concatenate(cols, axis=1)


def _post_body(q_ref, wpost_ref, bpost_ref, out_ref):
    q = q_ref[...].astype(jnp.bfloat16)
    out_ref[...] = (jnp.dot(q, wpost_ref[...],
                            preferred_element_type=jnp.float32)
                    + bpost_ref[...])


def kernel(x, W_pre, b_pre, W_wp, b_wp, codebook, W_post, b_post):
    B, T, IN = x.shape
    BT = B * T
    OUT = W_post.shape[1]
    GV = G * V
    NROWS = BT * G

    R = 2304
    x2 = x.reshape(BT, IN)
    idx = pl.pallas_call(
        _idx_body,
        grid=(BT // R,),
        in_specs=[
            pl.BlockSpec((R, IN), lambda i: (i, 0)),
            pl.BlockSpec((IN, P), lambda i: (0, 0)),
            pl.BlockSpec((1, P), lambda i: (0, 0)),
            pl.BlockSpec((P, GV), lambda i: (0, 0)),
            pl.BlockSpec((1, GV), lambda i: (0, 0)),
        ],
        out_specs=pl.BlockSpec((R, G), lambda i: (i, 0)),
        out_shape=jax.ShapeDtypeStruct((BT, G), jnp.int32),
    )(x2, W_pre, b_pre.reshape(1, P), W_wp, b_wp.reshape(1, GV))
    idx_flat = idx.reshape(NROWS)

    NW = 32
    RPW = NROWS // NW
    CH = 128
    NCH = RPW // CH

    cb128 = jnp.pad(codebook, ((0, 0), (0, 128 - D)))

    @functools.partial(
        pl.kernel,
        mesh=plsc.VectorSubcoreMesh(core_axis_name="c", subcore_axis_name="s"),
        out_type=jax.ShapeDtypeStruct((NROWS, 128), jnp.float32),
        scratch_types=[
            pltpu.VMEM((NCH, CH), jnp.int32),
            pltpu.VMEM((CH, 128), jnp.float32),
            pltpu.VMEM((CH, 128), jnp.float32),
            pltpu.SemaphoreType.DMA,
            pltpu.SemaphoreType.DMA,
        ],
    )
    def _gather(cb_hbm, idx_hbm, out_hbm, idx_all, rows0, rows1, sem0, sem1):
        wid = lax.axis_index("s") * 2 + lax.axis_index("c")
        base = wid * RPW
        pltpu.sync_copy(idx_hbm.at[wid], idx_all)
        bufs, sems = (rows0, rows1), (sem0, sem1)
        copies = [pltpu.async_copy(cb_hbm.at[idx_all.at[c]],
                                   bufs[c % 2], sems[c % 2])
                  for c in range(min(2, NCH))]
        for c in range(NCH):
            copies[c].wait()
            pltpu.sync_copy(bufs[c % 2], out_hbm.at[pl.ds(base + c * CH, CH)])
            if c + 2 < NCH:
                copies.append(pltpu.async_copy(cb_hbm.at[idx_all.at[c + 2]],
                                               bufs[c % 2], sems[c % 2]))

    q2 = _gather(cb128, idx_flat.reshape(NW, NCH, CH))

    q = q2.reshape(BT, G * 128)
    wpost_pad = jnp.pad(W_post.reshape(G, D, OUT), ((0, 0), (0, 128 - D), (0, 0))
                        ).reshape(G * 128, OUT)
    out = pl.pallas_call(
        _post_body,
        grid=(BT // R,),
        in_specs=[
            pl.BlockSpec((R, G * 128), lambda i: (i, 0)),
            pl.BlockSpec((G * 128, OUT), lambda i: (0, 0)),
            pl.BlockSpec((1, OUT), lambda i: (0, 0)),
        ],
        out_specs=pl.BlockSpec((R, OUT), lambda i: (i, 0)),
        out_shape=jax.ShapeDtypeStruct((BT, OUT), jnp.float32),
    )(q, wpost_pad.astype(jnp.bfloat16), b_post.reshape(1, OUT))
    return out.reshape(B, T, OUT)


# speedup vs baseline: 5.9711x; 5.9418x over previous
import jax
import jax.numpy as jnp
from jax.experimental import pallas as pl
from jax.experimental.pallas import tpu as pltpu

G, V, D, P = 8, 64, 64, 32


def _main_body(x_ref, wpre_ref, bpre_ref, wwp_ref, bwp_ref, cb_ref,
               wpost_ref, bpost_ref, out_ref, m_ref):
    @pl.when(pl.program_id(0) == 0)
    def _fuse_table():
        for g in range(G):
            m_ref[g * V:(g + 1) * V, :] = jnp.dot(
                cb_ref[g * V:(g + 1) * V, :],
                wpost_ref[g * V:(g + 1) * V, :],
                preferred_element_type=jnp.float32).astype(jnp.bfloat16)

    h = jnp.dot(x_ref[...], wpre_ref[...]) + bpre_ref[...]
    logits = jnp.dot(h, wwp_ref[...]) + bwp_ref[...]
    ohs = []
    for g in range(G):
        lg = logits[:, g * V:(g + 1) * V]
        mx = jnp.max(lg, axis=1, keepdims=True)
        ohs.append(jnp.where(lg >= mx, 1.0, 0.0))
    oh = jnp.concatenate(ohs, axis=1).astype(jnp.bfloat16)
    out_ref[...] = (jnp.dot(oh, m_ref[...],
                            preferred_element_type=jnp.float32)
                    + bpost_ref[...])


def kernel(x, W_pre, b_pre, W_wp, b_wp, codebook, W_post, b_post):
    B, T, IN = x.shape
    BT = B * T
    OUT = W_post.shape[1]
    GV = G * V

    R = 2304
    x2 = x.reshape(BT, IN)
    out = pl.pallas_call(
        _main_body,
        grid=(BT // R,),
        in_specs=[
            pl.BlockSpec((R, IN), lambda i: (i, 0)),
            pl.BlockSpec((IN, P), lambda i: (0, 0)),
            pl.BlockSpec((1, P), lambda i: (0, 0)),
            pl.BlockSpec((P, GV), lambda i: (0, 0)),
            pl.BlockSpec((1, GV), lambda i: (0, 0)),
            pl.BlockSpec((GV, D), lambda i: (0, 0)),
            pl.BlockSpec((GV, OUT), lambda i: (0, 0)),
            pl.BlockSpec((1, OUT), lambda i: (0, 0)),
        ],
        out_specs=pl.BlockSpec((R, OUT), lambda i: (i, 0)),
        out_shape=jax.ShapeDtypeStruct((BT, OUT), jnp.float32),
        scratch_shapes=[pltpu.VMEM((GV, OUT), jnp.bfloat16)],
    )(x2, W_pre, b_pre.reshape(1, P), W_wp, b_wp.reshape(1, GV), codebook,
      W_post, b_post.reshape(1, OUT))
    return out.reshape(B, T, OUT)
